# baseline (speedup 1.0000x reference)
import functools

import jax
import jax.numpy as jnp
from jax.experimental import pallas as pl
from jax.experimental.pallas import tpu as pltpu

B, S, D = 64, 4096, 128
BLK = 256
GROUP = 128
NC = GROUP // 8
P_MAX = 500.0
P_MIN = 0.0


def _body(params_ref, solar_ref, x_ref, w_ref, o_ref):
    b = pl.program_id(0)
    xv = x_ref[0]
    sol = solar_ref[b, 0]

    @pl.when(sol == 1)
    def _():
        coef = params_ref[0, 0]
        scale = params_ref[0, 1]
        w = w_ref[0]
        f = coef * jnp.clip(w * scale, 0.01, 1.0)
        xv = x_ref[0]
        a = xv * f
        t = jnp.sum(a, axis=1, keepdims=True)
        mx = jnp.max(a, axis=1, keepdims=True)
        mn = jnp.min(a, axis=1, keepdims=True)
        rng = jnp.maximum(mx - mn, 1.0)
        mid = jnp.zeros_like(t)
        step = 0.5 * rng
        for _ in range(5):
            tot = jnp.sum(jnp.clip(a - mid, P_MIN, P_MAX),
                          axis=1, keepdims=True)
            diff = tot - t
            delta = jnp.where(diff >= 0.1, step,
                              jnp.where(diff <= -0.1, -step, 0.0))
            mid = mid + delta
            step = step * 0.5
        o_ref[0] = jnp.clip(a - mid, P_MIN, P_MAX)

    @pl.when(sol != 1)
    def _():
        o_ref[0] = jnp.maximum(xv, 0.0)


@jax.jit
def _run(x, w3, solar, params):
    grid = (B, S // BLK)
    return pl.pallas_call(
        _body,
        grid=grid,
        in_specs=[
            pl.BlockSpec(memory_space=pltpu.SMEM),
            pl.BlockSpec(memory_space=pltpu.SMEM),
            pl.BlockSpec((1, BLK, D), lambda b, s: (b, s, 0)),
            pl.BlockSpec((1, BLK, 1), lambda b, s: (b, s, 0)),
        ],
        out_specs=pl.BlockSpec((1, BLK, D), lambda b, s: (b, s, 0)),
        out_shape=jax.ShapeDtypeStruct((B, S, D), jnp.float32),
        compiler_params=pltpu.CompilerParams(
            dimension_semantics=("parallel", "parallel"),
        ),
    )(params, solar, x, w3)


def kernel(x, weather_data, is_solar, unit_ids, c_prime, alpha, alpha_prime,
           ssrd_scale, A, eta):
    coef = c_prime * A * eta / (alpha + alpha_prime) * P_MAX
    params = jnp.stack([coef, ssrd_scale]).reshape(1, 2).astype(jnp.float32)
    w3 = weather_data.reshape(B, S, 1)
    return _run(x, w3, is_solar, params)


# speedup vs baseline: 2.2390x; 2.2390x over previous
import functools

import jax
import jax.numpy as jnp
from jax.experimental import pallas as pl
from jax.experimental.pallas import tpu as pltpu

B, S, D = 64, 4096, 128
BLK = 2048
GROUP = 128
NC = GROUP // 8
P_MAX = 500.0
P_MIN = 0.0


def _body(params_ref, solar_ref, x_ref, w_ref, o_ref):
    b = pl.program_id(0)
    xv = x_ref[0]
    sol = solar_ref[b, 0]

    @pl.when(sol == 1)
    def _():
        coef = params_ref[0, 0]
        scale = params_ref[0, 1]
        w = w_ref[0]
        f = coef * jnp.clip(w * scale, 0.01, 1.0)
        xv = x_ref[0]
        a = xv * f
        t = jnp.sum(a, axis=1, keepdims=True)
        mx = jnp.max(a, axis=1, keepdims=True)
        mn = jnp.min(a, axis=1, keepdims=True)
        rng = jnp.maximum(mx - mn, 1.0)
        mid = jnp.zeros_like(t)
        step = 0.5 * rng
        for _ in range(5):
            tot = jnp.sum(jnp.clip(a - mid, P_MIN, P_MAX),
                          axis=1, keepdims=True)
            diff = tot - t
            delta = jnp.where(diff >= 0.1, step,
                              jnp.where(diff <= -0.1, -step, 0.0))
            mid = mid + delta
            step = step * 0.5
        o_ref[0] = jnp.clip(a - mid, P_MIN, P_MAX)

    @pl.when(sol != 1)
    def _():
        o_ref[0] = jnp.maximum(xv, 0.0)


@jax.jit
def _run(x, w3, solar, params):
    grid = (B, S // BLK)
    return pl.pallas_call(
        _body,
        grid=grid,
        in_specs=[
            pl.BlockSpec(memory_space=pltpu.SMEM),
            pl.BlockSpec(memory_space=pltpu.SMEM),
            pl.BlockSpec((1, BLK, D), lambda b, s: (b, s, 0)),
            pl.BlockSpec((1, BLK, 1), lambda b, s: (b, s, 0)),
        ],
        out_specs=pl.BlockSpec((1, BLK, D), lambda b, s: (b, s, 0)),
        out_shape=jax.ShapeDtypeStruct((B, S, D), jnp.float32),
        compiler_params=pltpu.CompilerParams(
            dimension_semantics=("parallel", "parallel"),
        ),
    )(params, solar, x, w3)


def kernel(x, weather_data, is_solar, unit_ids, c_prime, alpha, alpha_prime,
           ssrd_scale, A, eta):
    coef = c_prime * A * eta / (alpha + alpha_prime) * P_MAX
    params = jnp.stack([coef, ssrd_scale]).reshape(1, 2).astype(jnp.float32)
    w3 = weather_data.reshape(B, S, 1)
    return _run(x, w3, is_solar, params)


# speedup vs baseline: 2.2843x; 1.0202x over previous
import functools

import jax
import jax.numpy as jnp
from jax.experimental import pallas as pl
from jax.experimental.pallas import tpu as pltpu

B, S, D = 64, 4096, 128
BLK = 4096
GROUP = 128
NC = GROUP // 8
P_MAX = 500.0
P_MIN = 0.0


def _body(params_ref, solar_ref, x_ref, w_ref, o_ref):
    b = pl.program_id(0)
    xv = x_ref[0]
    sol = solar_ref[b, 0]

    @pl.when(sol == 1)
    def _():
        coef = params_ref[0, 0]
        scale = params_ref[0, 1]
        w = w_ref[0]
        f = coef * jnp.clip(w * scale, 0.01, 1.0)
        xv = x_ref[0]
        a = xv * f
        t = jnp.sum(a, axis=1, keepdims=True)
        mx = jnp.max(a, axis=1, keepdims=True)
        mn = jnp.min(a, axis=1, keepdims=True)
        rng = jnp.maximum(mx - mn, 1.0)
        mid = jnp.zeros_like(t)
        step = 0.5 * rng
        for _ in range(5):
            tot = jnp.sum(jnp.clip(a - mid, P_MIN, P_MAX),
                          axis=1, keepdims=True)
            diff = tot - t
            delta = jnp.where(diff >= 0.1, step,
                              jnp.where(diff <= -0.1, -step, 0.0))
            mid = mid + delta
            step = step * 0.5
        o_ref[0] = jnp.clip(a - mid, P_MIN, P_MAX)

    @pl.when(sol != 1)
    def _():
        o_ref[0] = jnp.maximum(xv, 0.0)


@jax.jit
def _run(x, w3, solar, params):
    grid = (B, S // BLK)
    return pl.pallas_call(
        _body,
        grid=grid,
        in_specs=[
            pl.BlockSpec(memory_space=pltpu.SMEM),
            pl.BlockSpec(memory_space=pltpu.SMEM),
            pl.BlockSpec((1, BLK, D), lambda b, s: (b, s, 0)),
            pl.BlockSpec((1, BLK, 1), lambda b, s: (b, s, 0)),
        ],
        out_specs=pl.BlockSpec((1, BLK, D), lambda b, s: (b, s, 0)),
        out_shape=jax.ShapeDtypeStruct((B, S, D), jnp.float32),
        compiler_params=pltpu.CompilerParams(
            dimension_semantics=("parallel", "parallel"),
        ),
    )(params, solar, x, w3)


def kernel(x, weather_data, is_solar, unit_ids, c_prime, alpha, alpha_prime,
           ssrd_scale, A, eta):
    coef = c_prime * A * eta / (alpha + alpha_prime) * P_MAX
    params = jnp.stack([coef, ssrd_scale]).reshape(1, 2).astype(jnp.float32)
    w3 = weather_data.reshape(B, S, 1)
    return _run(x, w3, is_solar, params)


# speedup vs baseline: 2.2859x; 1.0007x over previous
import functools

import jax
import jax.numpy as jnp
from jax.experimental import pallas as pl
from jax.experimental.pallas import tpu as pltpu

B, S, D = 64, 4096, 128
BLK = 4096
GROUP = 128
NC = GROUP // 8
P_MAX = 500.0
P_MIN = 0.0


def _body(params_ref, solar_ref, x_ref, w_ref, o_ref):
    b = pl.program_id(0)
    xv = x_ref[0]
    sol = solar_ref[b, 0]

    @pl.when(sol == 1)
    def _():
        coef = params_ref[0, 0]
        scale = params_ref[0, 1]
        w = w_ref[0]
        f = coef * jnp.clip(w * scale, 0.01, 1.0)
        xv = x_ref[0]
        a = xv * f
        t = jnp.sum(a, axis=1)
        mx = jnp.max(a, axis=1)
        mn = jnp.min(a, axis=1)
        rng = jnp.maximum(mx - mn, 1.0)
        mid = jnp.zeros_like(t)
        for k in range(5):
            tot = jnp.sum(jnp.clip(a - mid[:, None], P_MIN, P_MAX), axis=1)
            diff = tot - t
            s = rng * (0.5 ** (k + 1))
            delta = jnp.where(diff >= 0.1, s,
                              jnp.where(diff <= -0.1, -s, 0.0))
            mid = mid + delta
        o_ref[0] = jnp.clip(a - mid[:, None], P_MIN, P_MAX)

    @pl.when(sol != 1)
    def _():
        o_ref[0] = jnp.maximum(xv, 0.0)


@jax.jit
def _run(x, w3, solar, params):
    grid = (B, S // BLK)
    return pl.pallas_call(
        _body,
        grid=grid,
        in_specs=[
            pl.BlockSpec(memory_space=pltpu.SMEM),
            pl.BlockSpec(memory_space=pltpu.SMEM),
            pl.BlockSpec((1, BLK, D), lambda b, s: (b, s, 0)),
            pl.BlockSpec((1, BLK, 1), lambda b, s: (b, s, 0)),
        ],
        out_specs=pl.BlockSpec((1, BLK, D), lambda b, s: (b, s, 0)),
        out_shape=jax.ShapeDtypeStruct((B, S, D), jnp.float32),
        compiler_params=pltpu.CompilerParams(
            dimension_semantics=("parallel", "parallel"),
        ),
    )(params, solar, x, w3)


def kernel(x, weather_data, is_solar, unit_ids, c_prime, alpha, alpha_prime,
           ssrd_scale, A, eta):
    coef = c_prime * A * eta / (alpha + alpha_prime) * P_MAX
    params = jnp.stack([coef, ssrd_scale]).reshape(1, 2).astype(jnp.float32)
    w3 = weather_data.reshape(B, S, 1)
    return _run(x, w3, is_solar, params)
